# Initial kernel scaffold; baseline (speedup 1.0000x reference)
import functools

import jax
import jax.numpy as jnp
from jax import lax
from jax.experimental import pallas as pl
from jax.experimental.pallas import tpu as pltpu
from jax.experimental.pallas import tpu_sc as plsc

G = 64
GA = 72
CHUNK = 128
BLK = 512


def _sc_geometry():
    info = plsc.get_sparse_core_info()
    return info.num_cores, info.num_subcores


def _make_deg_kernel(NC, NS, NP, CH):
    NW = NC * NS
    RPW = NP // NW
    mesh = plsc.VectorSubcoreMesh(core_axis_name="c", subcore_axis_name="s")

    @functools.partial(
        pl.kernel,
        out_type=jax.ShapeDtypeStruct((NC, NP), jnp.float32),
        mesh=mesh,
        scratch_types=[
            pltpu.VMEM_SHARED((NP,), jnp.float32),
            pltpu.VMEM((CH, CHUNK), jnp.int32),
            pltpu.VMEM((CHUNK,), jnp.float32),
            pltpu.SemaphoreType.DMA,
        ],
    )
    def deg_kernel(dst_hbm, ones_hbm, zeros_hbm, out_hbm, acc, dst_v, ones_v, sem):
        c = lax.axis_index("c")
        s = lax.axis_index("s")
        w = s * NC + c
        base = w * RPW
        pltpu.sync_copy(zeros_hbm.at[pl.ds(base, RPW)], acc.at[pl.ds(base, RPW)])
        pltpu.sync_copy(ones_hbm, ones_v)
        pltpu.sync_copy(dst_hbm.at[w], dst_v)
        plsc.subcore_barrier()

        def body(ch, carry):
            pltpu.sync_copy(ones_v, acc.at[dst_v.at[ch]], add=True)
            return carry

        lax.fori_loop(0, CH, body, 0, unroll=False)
        plsc.subcore_barrier()
        pltpu.sync_copy(acc.at[pl.ds(base, RPW)], out_hbm.at[c, pl.ds(base, RPW)])

    return deg_kernel


def _make_msgpass_kernel(NC, NS, NP, CH):
    NW = NC * NS
    RPW = NP // NW
    mesh = plsc.VectorSubcoreMesh(core_axis_name="c", subcore_axis_name="s")

    @functools.partial(
        pl.kernel,
        out_type=jax.ShapeDtypeStruct((NC, NP, 128), jnp.float32),
        mesh=mesh,
        scratch_types=[
            pltpu.VMEM_SHARED((NP, 128), jnp.float32),
            pltpu.VMEM((CH, CHUNK), jnp.int32),
            pltpu.VMEM((CH, CHUNK), jnp.int32),
            pltpu.VMEM((CHUNK, 128), jnp.float32),
            pltpu.SemaphoreType.DMA,
        ],
    )
    def msg_kernel(xs_hbm, src_hbm, dst_hbm, zeros_hbm, out_hbm,
                   acc, src_v, dst_v, rows_v, sem):
        c = lax.axis_index("c")
        s = lax.axis_index("s")
        w = s * NC + c
        base = w * RPW
        @pl.when(c == 0)
        def _():
            pltpu.sync_copy(xs_hbm.at[pl.ds(base, RPW)], acc.at[pl.ds(base, RPW)])

        @pl.when(c != 0)
        def _():
            pltpu.sync_copy(zeros_hbm.at[pl.ds(base, RPW)], acc.at[pl.ds(base, RPW)])

        pltpu.sync_copy(src_hbm.at[w], src_v)
        pltpu.sync_copy(dst_hbm.at[w], dst_v)
        plsc.subcore_barrier()

        def body(ch, carry):
            pltpu.async_copy(xs_hbm.at[src_v.at[ch]], rows_v, sem).wait()
            pltpu.sync_copy(rows_v, acc.at[dst_v.at[ch]], add=True)
            return carry

        lax.fori_loop(0, CH, body, 0, unroll=False)
        plsc.subcore_barrier()
        pltpu.sync_copy(acc.at[pl.ds(base, RPW)], out_hbm.at[c, pl.ds(base, RPW)])

    return msg_kernel


def _make_pool_kernel(NC, NS, NP, CB):
    NW = NC * NS
    RPW = NP // NW
    mesh = plsc.VectorSubcoreMesh(core_axis_name="c", subcore_axis_name="s")

    @functools.partial(
        pl.kernel,
        out_type=(jax.ShapeDtypeStruct((NC, GA, 128), jnp.float32),
                  jax.ShapeDtypeStruct((NC, GA, 128), jnp.float32)),
        mesh=mesh,
        scratch_types=[
            pltpu.VMEM_SHARED((GA, 128), jnp.float32),
            pltpu.VMEM_SHARED((GA, 128), jnp.float32),
            pltpu.VMEM((CB, CHUNK), jnp.int32),
            pltpu.VMEM((CHUNK, 128), jnp.float32),
            pltpu.VMEM((CHUNK, 128), jnp.float32),
            pltpu.SemaphoreType.DMA,
        ],
    )
    def pool_kernel(h_hbm, batch_hbm, ones_hbm, zeros_hbm, sums_hbm, cnt_hbm,
                    sums_acc, cnt_acc, b_v, rows_v, ones_v, sem):
        c = lax.axis_index("c")
        s = lax.axis_index("s")
        w = s * NC + c

        @pl.when(s == 0)
        def _():
            pltpu.sync_copy(zeros_hbm.at[pl.ds(0, GA)], sums_acc)
            pltpu.sync_copy(zeros_hbm.at[pl.ds(0, GA)], cnt_acc)

        pltpu.sync_copy(ones_hbm, ones_v)
        pltpu.sync_copy(batch_hbm.at[w], b_v)
        plsc.subcore_barrier()

        def body(ch, carry):
            pltpu.sync_copy(h_hbm.at[pl.ds(w * RPW + ch * CHUNK, CHUNK)], rows_v)
            pltpu.sync_copy(rows_v, sums_acc.at[b_v.at[ch]], add=True)
            pltpu.sync_copy(ones_v, cnt_acc.at[b_v.at[ch]], add=True)
            return carry

        lax.fori_loop(0, CB, body, 0, unroll=False)
        plsc.subcore_barrier()

        @pl.when(s == 0)
        def _():
            pltpu.sync_copy(sums_acc, sums_hbm.at[c])
            pltpu.sync_copy(cnt_acc, cnt_hbm.at[c])

    return pool_kernel


def _tc_pre_body(x_ref, Win_ref, bin_ref, conv0_ref, d0_ref, d1_ref,
                 h_ref, xs_ref, dinv_ref):
    deg = 1.0 + d0_ref[...] + d1_ref[...]
    dinv = lax.rsqrt(deg)
    x1 = jax.nn.relu(
        jnp.dot(x_ref[...], Win_ref[...], preferred_element_type=jnp.float32)
        + bin_ref[...])
    xw = jnp.dot(x1, conv0_ref[...], preferred_element_type=jnp.float32)
    h_ref[...] = x1
    xs_ref[...] = xw * dinv
    dinv_ref[...] = dinv


def _tc_layer_body(has_next, p0_ref, p1_ref, dinv_ref, h_ref, convb_ref,
                   WihT_ref, bih_ref, WhhT_ref, bhh_ref, convn_ref,
                   hout_ref, xsout_ref):
    H = 128
    dinv = dinv_ref[...]
    xc = jax.nn.relu(dinv * (p0_ref[...] + p1_ref[...]) + convb_ref[...])
    h = h_ref[...]
    gi = jnp.dot(xc, WihT_ref[...], preferred_element_type=jnp.float32) + bih_ref[...]
    gh = jnp.dot(h, WhhT_ref[...], preferred_element_type=jnp.float32) + bhh_ref[...]
    r = jax.nn.sigmoid(gi[:, 0:H] + gh[:, 0:H])
    z = jax.nn.sigmoid(gi[:, H:2 * H] + gh[:, H:2 * H])
    n = jnp.tanh(gi[:, 2 * H:3 * H] + r * gh[:, 2 * H:3 * H])
    hn = (1.0 - z) * n + z * h
    hout_ref[...] = hn
    if has_next:
        xw = jnp.dot(hn, convn_ref[...], preferred_element_type=jnp.float32)
        xsout_ref[...] = xw * dinv


def _tc_final_body(s0_ref, s1_ref, c0_ref, c1_ref, Wout_ref, bout_ref, out_ref):
    sums = s0_ref[...] + s1_ref[...]
    cnt = jnp.clip(c0_ref[...] + c1_ref[...], 1.0, None)
    gs = sums[0:G, :] / cnt[0:G, :]
    out_ref[...] = (
        jnp.dot(gs, Wout_ref[...], preferred_element_type=jnp.float32)
        + bout_ref[...])


def _full_spec(shape):
    return pl.BlockSpec(shape, lambda i: tuple(0 for _ in shape))


def _row_spec(cols):
    return pl.BlockSpec((BLK, cols), lambda i: (i, 0))


def kernel(x, edge_index, batch, W_in, b_in, convW, convb, W_ih, W_hh,
           b_ih, b_hh, W_out, b_out):
    N, D = x.shape
    E = edge_index.shape[1]
    H = W_in.shape[1]
    L = convW.shape[0]
    NC, NS = _sc_geometry()
    NW = NC * NS

    EW = NW * CHUNK
    E_pad = ((E + EW - 1) // EW) * EW
    CH = E_pad // EW
    NP = ((N + EW - 1) // EW) * EW
    CB = NP // EW

    src = edge_index[0]
    dst = edge_index[1]
    pad_e = E_pad - E
    src_p = jnp.concatenate(
        [src, jnp.zeros((pad_e,), jnp.int32)]).reshape(NW, CH, CHUNK)
    dst_p = jnp.concatenate(
        [dst, jnp.full((pad_e,), N, jnp.int32)]).reshape(NW, CH, CHUNK)
    batch_p = jnp.concatenate(
        [batch.astype(jnp.int32),
         jnp.full((NP - N,), G, jnp.int32)]).reshape(NW, CB, CHUNK)
    x_p = jnp.concatenate(
        [x, jnp.zeros((NP - N, D), jnp.float32)], axis=0)
    ones1 = jnp.ones((CHUNK,), jnp.float32)
    ones2 = jnp.ones((CHUNK, 128), jnp.float32)
    zeros1 = jnp.zeros((NP,), jnp.float32)
    zeros2 = jnp.zeros((NP, 128), jnp.float32)
    WihT = W_ih.T
    WhhT = W_hh.T

    n_blocks = NP // BLK

    degp = _make_deg_kernel(NC, NS, NP, CH)(dst_p, ones1, zeros1)
    d0 = degp[0].reshape(NP, 1)
    d1 = degp[1].reshape(NP, 1)

    h, xs, dinv = pl.pallas_call(
        _tc_pre_body,
        grid=(n_blocks,),
        in_specs=[
            _row_spec(D),
            _full_spec((D, H)),
            _full_spec((1, H)),
            _full_spec((H, H)),
            pl.BlockSpec((BLK, 1), lambda i: (i, 0)),
            pl.BlockSpec((BLK, 1), lambda i: (i, 0)),
        ],
        out_specs=[_row_spec(H), _row_spec(H),
                   pl.BlockSpec((BLK, 1), lambda i: (i, 0))],
        out_shape=[
            jax.ShapeDtypeStruct((NP, H), jnp.float32),
            jax.ShapeDtypeStruct((NP, H), jnp.float32),
            jax.ShapeDtypeStruct((NP, 1), jnp.float32),
        ],
    )(x_p, W_in, b_in.reshape(1, H), convW[0], d0, d1)

    msg = _make_msgpass_kernel(NC, NS, NP, CH)
    for l in range(L):
        p = msg(xs, src_p, dst_p, zeros2)
        has_next = l + 1 < L
        convn = convW[l + 1] if has_next else convW[l]
        outs = pl.pallas_call(
            functools.partial(_tc_layer_body, has_next),
            grid=(n_blocks,),
            in_specs=[
                _row_spec(H), _row_spec(H),
                pl.BlockSpec((BLK, 1), lambda i: (i, 0)),
                _row_spec(H),
                _full_spec((1, H)),
                _full_spec((H, 3 * H)),
                _full_spec((1, 3 * H)),
                _full_spec((H, 3 * H)),
                _full_spec((1, 3 * H)),
                _full_spec((H, H)),
            ],
            out_specs=[_row_spec(H), _row_spec(H)],
            out_shape=[
                jax.ShapeDtypeStruct((NP, H), jnp.float32),
                jax.ShapeDtypeStruct((NP, H), jnp.float32),
            ],
        )(p[0], p[1], dinv, h, convb[l].reshape(1, H), WihT,
          b_ih.reshape(1, 3 * H), WhhT, b_hh.reshape(1, 3 * H), convn)
        h = outs[0]
        if has_next:
            xs = outs[1]

    sums, cnts = _make_pool_kernel(NC, NS, NP, CB)(h, batch_p, ones2, zeros2)

    out = pl.pallas_call(
        _tc_final_body,
        grid=(1,),
        in_specs=[
            _full_spec((GA, 128)), _full_spec((GA, 128)),
            _full_spec((GA, 128)), _full_spec((GA, 128)),
            _full_spec((H, W_out.shape[1])),
            _full_spec((1, W_out.shape[1])),
        ],
        out_specs=_full_spec((G, W_out.shape[1])),
        out_shape=jax.ShapeDtypeStruct((G, W_out.shape[1]), jnp.float32),
    )(sums[0], sums[1], cnts[0], cnts[1], W_out,
      b_out.reshape(1, W_out.shape[1]))
    return out


# speedup vs baseline: 6.5240x; 6.5240x over previous
import functools

import jax
import jax.numpy as jnp
from jax import lax
from jax.experimental import pallas as pl
from jax.experimental.pallas import tpu as pltpu
from jax.experimental.pallas import tpu_sc as plsc

G = 64
GA = 72
CHUNK = 128
IB = 16
BLK = 512


def _sc_geometry():
    info = plsc.get_sparse_core_info()
    return info.num_cores, info.num_subcores


def _make_deg_kernel(NC, NS, NP, CH):
    NW = NC * NS
    RPS = NP // NS
    mesh = plsc.VectorSubcoreMesh(core_axis_name="c", subcore_axis_name="s")

    @functools.partial(
        pl.kernel,
        out_type=jax.ShapeDtypeStruct((NC, NP), jnp.float32),
        mesh=mesh,
        scratch_types=[
            pltpu.VMEM_SHARED((NP,), jnp.float32),
            pltpu.VMEM((CH, CHUNK), jnp.int32),
            pltpu.VMEM((CHUNK,), jnp.float32),
            pltpu.SemaphoreType.DMA,
        ],
    )
    def deg_kernel(dst_hbm, ones_hbm, zeros_hbm, out_hbm, acc, dst_v, ones_v, sem):
        c = lax.axis_index("c")
        s = lax.axis_index("s")
        w = s * NC + c
        base = s * RPS
        pltpu.sync_copy(zeros_hbm.at[pl.ds(base, RPS)], acc.at[pl.ds(base, RPS)])
        pltpu.sync_copy(ones_hbm, ones_v)
        pltpu.sync_copy(dst_hbm.at[w], dst_v)
        plsc.subcore_barrier()

        def body(ch, carry):
            pltpu.sync_copy(ones_v, acc.at[dst_v.at[ch]], add=True)
            return carry

        lax.fori_loop(0, CH, body, 0, unroll=False)
        plsc.subcore_barrier()
        pltpu.sync_copy(acc.at[pl.ds(base, RPS)], out_hbm.at[c, pl.ds(base, RPS)])

    return deg_kernel


def _make_msgpass_kernel(NC, NS, NP, CH):
    NW = NC * NS
    RPS = NP // NS
    mesh = plsc.VectorSubcoreMesh(core_axis_name="c", subcore_axis_name="s")

    @functools.partial(
        pl.kernel,
        out_type=jax.ShapeDtypeStruct((NC, NP, 128), jnp.float32),
        mesh=mesh,
        scratch_types=[
            pltpu.VMEM_SHARED((NP, 128), jnp.float32),
            pltpu.VMEM((IB, CHUNK), jnp.int32),
            pltpu.VMEM((IB, CHUNK), jnp.int32),
            pltpu.VMEM((CHUNK, 128), jnp.float32),
            pltpu.SemaphoreType.DMA,
        ],
    )
    def msg_kernel(xs_hbm, src_hbm, dst_hbm, zeros_hbm, out_hbm,
                   acc, src_v, dst_v, rows_v, sem):
        c = lax.axis_index("c")
        s = lax.axis_index("s")
        w = s * NC + c
        base = s * RPS
        @pl.when(c == 0)
        def _():
            pltpu.sync_copy(xs_hbm.at[pl.ds(base, RPS)], acc.at[pl.ds(base, RPS)])

        @pl.when(c != 0)
        def _():
            pltpu.sync_copy(zeros_hbm.at[pl.ds(base, RPS)], acc.at[pl.ds(base, RPS)])

        plsc.subcore_barrier()

        def group(g, carry):
            pltpu.sync_copy(src_hbm.at[w, pl.ds(g * IB, IB)], src_v)
            pltpu.sync_copy(dst_hbm.at[w, pl.ds(g * IB, IB)], dst_v)

            def body(ch, carry2):
                pltpu.async_copy(xs_hbm.at[src_v.at[ch]], rows_v, sem).wait()
                pltpu.sync_copy(rows_v, acc.at[dst_v.at[ch]], add=True)
                return carry2

            lax.fori_loop(0, IB, body, 0, unroll=False)
            return carry

        lax.fori_loop(0, CH // IB, group, 0, unroll=False)
        plsc.subcore_barrier()
        pltpu.sync_copy(acc.at[pl.ds(base, RPS)], out_hbm.at[c, pl.ds(base, RPS)])

    return msg_kernel


def _make_pool_kernel(NC, NS, NP, CB):
    NW = NC * NS
    RPW = NP // NW
    mesh = plsc.VectorSubcoreMesh(core_axis_name="c", subcore_axis_name="s")

    @functools.partial(
        pl.kernel,
        out_type=(jax.ShapeDtypeStruct((NC, GA, 128), jnp.float32),
                  jax.ShapeDtypeStruct((NC, GA, 128), jnp.float32)),
        mesh=mesh,
        scratch_types=[
            pltpu.VMEM_SHARED((GA, 128), jnp.float32),
            pltpu.VMEM_SHARED((GA, 128), jnp.float32),
            pltpu.VMEM((CB, CHUNK), jnp.int32),
            pltpu.VMEM((CHUNK, 128), jnp.float32),
            pltpu.VMEM((CHUNK, 128), jnp.float32),
            pltpu.SemaphoreType.DMA,
        ],
    )
    def pool_kernel(h_hbm, batch_hbm, ones_hbm, zeros_hbm, sums_hbm, cnt_hbm,
                    sums_acc, cnt_acc, b_v, rows_v, ones_v, sem):
        c = lax.axis_index("c")
        s = lax.axis_index("s")
        w = s * NC + c

        @pl.when(s == 0)
        def _():
            pltpu.sync_copy(zeros_hbm.at[pl.ds(0, GA)], sums_acc)
            pltpu.sync_copy(zeros_hbm.at[pl.ds(0, GA)], cnt_acc)

        pltpu.sync_copy(ones_hbm, ones_v)
        pltpu.sync_copy(batch_hbm.at[w], b_v)
        plsc.subcore_barrier()

        def body(ch, carry):
            pltpu.sync_copy(h_hbm.at[pl.ds(w * RPW + ch * CHUNK, CHUNK)], rows_v)
            pltpu.sync_copy(rows_v, sums_acc.at[b_v.at[ch]], add=True)
            pltpu.sync_copy(ones_v, cnt_acc.at[b_v.at[ch]], add=True)
            return carry

        lax.fori_loop(0, CB, body, 0, unroll=False)
        plsc.subcore_barrier()

        @pl.when(s == 0)
        def _():
            pltpu.sync_copy(sums_acc, sums_hbm.at[c])
            pltpu.sync_copy(cnt_acc, cnt_hbm.at[c])

    return pool_kernel


def _tc_pre_body(x_ref, Win_ref, bin_ref, conv0_ref, d0_ref, d1_ref,
                 h_ref, xs_ref, dinv_ref):
    deg = 1.0 + d0_ref[...] + d1_ref[...]
    dinv = lax.rsqrt(deg)
    x1 = jax.nn.relu(
        jnp.dot(x_ref[...], Win_ref[...], preferred_element_type=jnp.float32)
        + bin_ref[...])
    xw = jnp.dot(x1, conv0_ref[...], preferred_element_type=jnp.float32)
    h_ref[...] = x1
    xs_ref[...] = xw * dinv
    dinv_ref[...] = dinv


def _tc_layer_body(has_next, p0_ref, p1_ref, dinv_ref, h_ref, convb_ref,
                   WihT_ref, bih_ref, WhhT_ref, bhh_ref, convn_ref,
                   hout_ref, xsout_ref):
    H = 128
    dinv = dinv_ref[...]
    xc = jax.nn.relu(dinv * (p0_ref[...] + p1_ref[...]) + convb_ref[...])
    h = h_ref[...]
    gi = jnp.dot(xc, WihT_ref[...], preferred_element_type=jnp.float32) + bih_ref[...]
    gh = jnp.dot(h, WhhT_ref[...], preferred_element_type=jnp.float32) + bhh_ref[...]
    r = jax.nn.sigmoid(gi[:, 0:H] + gh[:, 0:H])
    z = jax.nn.sigmoid(gi[:, H:2 * H] + gh[:, H:2 * H])
    n = jnp.tanh(gi[:, 2 * H:3 * H] + r * gh[:, 2 * H:3 * H])
    hn = (1.0 - z) * n + z * h
    hout_ref[...] = hn
    if has_next:
        xw = jnp.dot(hn, convn_ref[...], preferred_element_type=jnp.float32)
        xsout_ref[...] = xw * dinv


def _tc_final_body(s0_ref, s1_ref, c0_ref, c1_ref, Wout_ref, bout_ref, out_ref):
    sums = s0_ref[...] + s1_ref[...]
    cnt = jnp.clip(c0_ref[...] + c1_ref[...], 1.0, None)
    gs = sums[0:G, :] / cnt[0:G, :]
    out_ref[...] = (
        jnp.dot(gs, Wout_ref[...], preferred_element_type=jnp.float32)
        + bout_ref[...])


def _full_spec(shape):
    return pl.BlockSpec(shape, lambda i: tuple(0 for _ in shape))


def _row_spec(cols):
    return pl.BlockSpec((BLK, cols), lambda i: (i, 0))


def kernel(x, edge_index, batch, W_in, b_in, convW, convb, W_ih, W_hh,
           b_ih, b_hh, W_out, b_out):
    N, D = x.shape
    E = edge_index.shape[1]
    H = W_in.shape[1]
    L = convW.shape[0]
    NC, NS = _sc_geometry()
    NW = NC * NS

    EW = NW * CHUNK
    EG = EW * IB
    E_pad = ((E + EG - 1) // EG) * EG
    CH = E_pad // EW
    NP = ((N + EW - 1) // EW) * EW
    CB = NP // EW

    src = edge_index[0]
    dst = edge_index[1]
    pad_e = E_pad - E
    src_p = jnp.concatenate(
        [src, jnp.zeros((pad_e,), jnp.int32)]).reshape(NW, CH, CHUNK)
    dst_p = jnp.concatenate(
        [dst, jnp.full((pad_e,), N, jnp.int32)]).reshape(NW, CH, CHUNK)
    batch_p = jnp.concatenate(
        [batch.astype(jnp.int32),
         jnp.full((NP - N,), G, jnp.int32)]).reshape(NW, CB, CHUNK)
    x_p = jnp.concatenate(
        [x, jnp.zeros((NP - N, D), jnp.float32)], axis=0)
    ones1 = jnp.ones((CHUNK,), jnp.float32)
    ones2 = jnp.ones((CHUNK, 128), jnp.float32)
    zeros1 = jnp.zeros((NP,), jnp.float32)
    zeros2 = jnp.zeros((NP, 128), jnp.float32)
    WihT = W_ih.T
    WhhT = W_hh.T

    n_blocks = NP // BLK

    degp = _make_deg_kernel(NC, NS, NP, CH)(dst_p, ones1, zeros1)
    d0 = degp[0].reshape(NP, 1)
    d1 = degp[1].reshape(NP, 1)

    h, xs, dinv = pl.pallas_call(
        _tc_pre_body,
        grid=(n_blocks,),
        in_specs=[
            _row_spec(D),
            _full_spec((D, H)),
            _full_spec((1, H)),
            _full_spec((H, H)),
            pl.BlockSpec((BLK, 1), lambda i: (i, 0)),
            pl.BlockSpec((BLK, 1), lambda i: (i, 0)),
        ],
        out_specs=[_row_spec(H), _row_spec(H),
                   pl.BlockSpec((BLK, 1), lambda i: (i, 0))],
        out_shape=[
            jax.ShapeDtypeStruct((NP, H), jnp.float32),
            jax.ShapeDtypeStruct((NP, H), jnp.float32),
            jax.ShapeDtypeStruct((NP, 1), jnp.float32),
        ],
    )(x_p, W_in, b_in.reshape(1, H), convW[0], d0, d1)

    msg = _make_msgpass_kernel(NC, NS, NP, CH)
    for l in range(L):
        p = msg(xs, src_p, dst_p, zeros2)
        has_next = l + 1 < L
        convn = convW[l + 1] if has_next else convW[l]
        outs = pl.pallas_call(
            functools.partial(_tc_layer_body, has_next),
            grid=(n_blocks,),
            in_specs=[
                _row_spec(H), _row_spec(H),
                pl.BlockSpec((BLK, 1), lambda i: (i, 0)),
                _row_spec(H),
                _full_spec((1, H)),
                _full_spec((H, 3 * H)),
                _full_spec((1, 3 * H)),
                _full_spec((H, 3 * H)),
                _full_spec((1, 3 * H)),
                _full_spec((H, H)),
            ],
            out_specs=[_row_spec(H), _row_spec(H)],
            out_shape=[
                jax.ShapeDtypeStruct((NP, H), jnp.float32),
                jax.ShapeDtypeStruct((NP, H), jnp.float32),
            ],
        )(p[0], p[1], dinv, h, convb[l].reshape(1, H), WihT,
          b_ih.reshape(1, 3 * H), WhhT, b_hh.reshape(1, 3 * H), convn)
        h = outs[0]
        if has_next:
            xs = outs[1]

    sums, cnts = _make_pool_kernel(NC, NS, NP, CB)(h, batch_p, ones2, zeros2)

    out = pl.pallas_call(
        _tc_final_body,
        grid=(1,),
        in_specs=[
            _full_spec((GA, 128)), _full_spec((GA, 128)),
            _full_spec((GA, 128)), _full_spec((GA, 128)),
            _full_spec((H, W_out.shape[1])),
            _full_spec((1, W_out.shape[1])),
        ],
        out_specs=_full_spec((G, W_out.shape[1])),
        out_shape=jax.ShapeDtypeStruct((G, W_out.shape[1]), jnp.float32),
    )(sums[0], sums[1], cnts[0], cnts[1], W_out,
      b_out.reshape(1, W_out.shape[1]))
    return out
